# Initial kernel scaffold; baseline (speedup 1.0000x reference)
import jax
import jax.numpy as jnp
from jax.experimental import pallas as pl


def kernel(logits):
    raise NotImplementedError("write your pallas kernel here")


# speedup vs baseline: 11.8759x; 11.8759x over previous
import functools

import jax
import jax.numpy as jnp
from jax import lax
from jax.experimental import pallas as pl
from jax.experimental.pallas import tpu as pltpu
from jax.experimental.pallas import tpu_sc as plsc

ROWS = 128
COLS = 100000
TOPK = 50
CHUNK = 128
NCHUNK = COLS // CHUNK
TAIL = COLS - NCHUNK * CHUNK
TAIL_START = NCHUNK * CHUNK
CMP = 784
CAPC = 64
CAP = 128

K1_BLOCK_ROWS = 8
K1_GRID = ROWS // K1_BLOCK_ROWS

NC = 2
NS = 16
NW = NC * NS
ROWS_PER_W = ROWS // NW


def _monotone_u32(x):
    b = lax.bitcast_convert_type(x, jnp.uint32)
    neg = (b >> jnp.uint32(31)) == jnp.uint32(1)
    return jnp.where(neg, ~b, b | jnp.uint32(0x80000000))


def _u32_to_f32(u):
    was_pos = (u >> jnp.uint32(31)) == jnp.uint32(1)
    b = jnp.where(was_pos, u & jnp.uint32(0x7FFFFFFF), ~u)
    return lax.bitcast_convert_type(b, jnp.float32)


def _kth_largest_key(ukey, k, batch):
    t = jnp.zeros((batch, 1), jnp.uint32)
    for bit in range(31, -1, -1):
        cand = t | jnp.uint32(1 << bit)
        cnt = jnp.sum((ukey >= cand).astype(jnp.int32), axis=1, keepdims=True)
        t = jnp.where(cnt >= k, cand, t)
    return t


def _threefry_gumbel_bits(flat_idx_u32):
    k1 = jnp.uint32(0)
    k2 = jnp.uint32(42)
    ks0, ks1 = k1, k2
    ks2 = k1 ^ k2 ^ jnp.uint32(0x1BD11BDA)

    rot1 = (13, 15, 26, 6)
    rot2 = (17, 29, 16, 24)

    def rounds(x0, x1, rots):
        for r in rots:
            x0 = x0 + x1
            x1 = (x1 << jnp.uint32(r)) | (x1 >> jnp.uint32(32 - r))
            x1 = x0 ^ x1
        return x0, x1

    x0 = jnp.zeros_like(flat_idx_u32) + ks0
    x1 = flat_idx_u32 + ks1
    x0, x1 = rounds(x0, x1, rot1)
    x0, x1 = x0 + ks1, x1 + ks2 + jnp.uint32(1)
    x0, x1 = rounds(x0, x1, rot2)
    x0, x1 = x0 + ks2, x1 + ks0 + jnp.uint32(2)
    x0, x1 = rounds(x0, x1, rot1)
    x0, x1 = x0 + ks0, x1 + ks1 + jnp.uint32(3)
    x0, x1 = rounds(x0, x1, rot2)
    x0, x1 = x0 + ks1, x1 + ks2 + jnp.uint32(4)
    x0, x1 = rounds(x0, x1, rot1)
    x0, x1 = x0 + ks2, x1 + ks0 + jnp.uint32(5)
    return x0 ^ x1


def _gumbel_from_bits(bits):
    fb = (bits >> jnp.uint32(9)) | jnp.uint32(0x3F800000)
    f = lax.bitcast_convert_type(fb, jnp.float32) - jnp.float32(1.0)
    tiny = jnp.float32(1.1754943508222875e-38)
    u = f * (jnp.float32(1.0) - tiny) + tiny
    u = jnp.maximum(tiny, u)
    return -jnp.log(-jnp.log(u))


def _stats_kernel(x_ref, cm_ref, t_ref):
    x = x_ref[...]
    xa = x[:, : NCHUNK * CHUNK].reshape(K1_BLOCK_ROWS, NCHUNK, CHUNK)
    cm = jnp.max(xa, axis=2)
    t = _kth_largest_key(_monotone_u32(cm), TOPK, K1_BLOCK_ROWS)
    tf = _u32_to_f32(t)
    pad = jnp.full((K1_BLOCK_ROWS, CMP - NCHUNK), -jnp.inf, jnp.float32)
    cm_ref[...] = jnp.concatenate([cm, pad], axis=1)
    t_ref[...] = jnp.broadcast_to(tf, (K1_BLOCK_ROWS, 16))


def _run_stats(logits):
    return pl.pallas_call(
        _stats_kernel,
        grid=(K1_GRID,),
        in_specs=[pl.BlockSpec((K1_BLOCK_ROWS, COLS), lambda i: (i, 0))],
        out_specs=[
            pl.BlockSpec((K1_BLOCK_ROWS, CMP), lambda i: (i, 0)),
            pl.BlockSpec((K1_BLOCK_ROWS, 16), lambda i: (i, 0)),
        ],
        out_shape=[
            jax.ShapeDtypeStruct((ROWS, CMP), jnp.float32),
            jax.ShapeDtypeStruct((ROWS, 16), jnp.float32),
        ],
    )(logits)


def _candidates_kernel(x_hbm, cm_hbm, t_hbm, cv_hbm, ci_hbm, nc_hbm,
                       cm_v, t_v, qc_v, chunks_v, cval_v, cidx_v, tail_v,
                       nc_v, sem):
    wid = lax.axis_index("s") * NC + lax.axis_index("c")
    lanes = lax.iota(jnp.int32, 16)

    for rr in range(ROWS_PER_W):
        r = wid * ROWS_PER_W + rr

        pltpu.sync_copy(cm_hbm.at[r], cm_v)
        pltpu.sync_copy(t_hbm.at[r], t_v)
        tvec = t_v[...]

        for g in range(CAPC // 16):
            qc_v[pl.ds(g * 16, 16)] = jnp.zeros((16,), jnp.int32)
        off = jnp.int32(0)
        for g in range(CMP // 16):
            v = cm_v[pl.ds(g * 16, 16)]
            m = v >= tvec
            ids = lanes + jnp.int32(g * 16)
            w = jnp.minimum(off, CAPC)
            plsc.store_compressed(qc_v.at[pl.ds(w, 16)], ids, mask=m)
            off = off + jnp.sum(m.astype(jnp.int32))
        nq = jnp.minimum(off, CAPC)

        qgroups = [qc_v[pl.ds(g * 16, 16)] for g in range(CAPC // 16)]
        for j in range(CAPC):
            c = jnp.clip(qgroups[j // 16][j % 16], 0, NCHUNK - 1)
            pltpu.async_copy(
                x_hbm.at[r, pl.ds(c * CHUNK, CHUNK)], chunks_v.at[j], sem)
        pltpu.async_copy(
            x_hbm.at[r, pl.ds(TAIL_START, TAIL)], tail_v, sem)
        for j in range(CAPC):
            pltpu.make_async_copy(
                x_hbm.at[r, pl.ds(0, CHUNK)], chunks_v.at[j], sem).wait()
        pltpu.make_async_copy(
            x_hbm.at[r, pl.ds(0, TAIL)], tail_v, sem).wait()

        def chunk_body(j, noff):
            jv = jnp.full((16,), j, jnp.int32)
            cid = plsc.load_gather(qc_v, [jv])
            base = cid * jnp.int32(CHUNK)
            for s in range(CHUNK // 16):
                vals = chunks_v[j, pl.ds(s * 16, 16)]
                m = vals >= tvec
                gidx = base + jnp.int32(s * 16) + lanes
                w = jnp.minimum(noff, CAP)
                plsc.store_compressed(cval_v.at[pl.ds(w, 16)], vals, mask=m)
                plsc.store_compressed(cidx_v.at[pl.ds(w, 16)], gidx, mask=m)
                noff = noff + jnp.sum(m.astype(jnp.int32))
            return noff

        noff = lax.fori_loop(0, nq, chunk_body, jnp.int32(0))

        for s in range(TAIL // 16):
            vals = tail_v[pl.ds(s * 16, 16)]
            m = vals >= tvec
            gidx = jnp.int32(TAIL_START + s * 16) + lanes
            w = jnp.minimum(noff, CAP)
            plsc.store_compressed(cval_v.at[pl.ds(w, 16)], vals, mask=m)
            plsc.store_compressed(cidx_v.at[pl.ds(w, 16)], gidx, mask=m)
            noff = noff + jnp.sum(m.astype(jnp.int32))

        nc_v[...] = jnp.broadcast_to(jnp.minimum(noff, CAP), (16,)).astype(
            jnp.int32)
        pltpu.sync_copy(cval_v.at[pl.ds(0, CAP)], cv_hbm.at[r])
        pltpu.sync_copy(cidx_v.at[pl.ds(0, CAP)], ci_hbm.at[r])
        pltpu.sync_copy(nc_v, nc_hbm.at[r])


def _run_candidates(logits, cm, t):
    mesh = plsc.VectorSubcoreMesh(core_axis_name="c", subcore_axis_name="s")
    fn = functools.partial(
        pl.kernel,
        mesh=mesh,
        compiler_params=pltpu.CompilerParams(needs_layout_passes=False),
        out_type=[
            jax.ShapeDtypeStruct((ROWS, CAP), jnp.float32),
            jax.ShapeDtypeStruct((ROWS, CAP), jnp.int32),
            jax.ShapeDtypeStruct((ROWS, 16), jnp.int32),
        ],
        scratch_types=[
            pltpu.VMEM((CMP,), jnp.float32),
            pltpu.VMEM((16,), jnp.float32),
            pltpu.VMEM((CAPC + 16,), jnp.int32),
            pltpu.VMEM((CAPC, CHUNK), jnp.float32),
            pltpu.VMEM((CAP + 16,), jnp.float32),
            pltpu.VMEM((CAP + 16,), jnp.int32),
            pltpu.VMEM((TAIL,), jnp.float32),
            pltpu.VMEM((16,), jnp.int32),
            pltpu.SemaphoreType.DMA,
        ],
    )(_candidates_kernel)
    return fn(logits, cm, t)


def _finish_kernel(cv_ref, ci_ref, nc_ref, out_ref):
    vals = cv_ref[...]
    idxs = ci_ref[...]
    nc = nc_ref[...][:, 0:1]
    slot = lax.broadcasted_iota(jnp.int32, (ROWS, CAP), 1)
    valid = slot < nc

    ukey = jnp.where(valid, _monotone_u32(vals), jnp.uint32(0))
    kth = _u32_to_f32(_kth_largest_key(ukey, TOPK, ROWS))
    keep = valid & (vals >= kth)

    rowi = lax.broadcasted_iota(jnp.int32, (ROWS, CAP), 0)
    flat = lax.bitcast_convert_type(rowi * COLS + idxs, jnp.uint32)
    g = _gumbel_from_bits(_threefry_gumbel_bits(flat))

    score = jnp.where(keep, vals + g, jnp.float32(-jnp.inf))
    win = jnp.argmax(score, axis=1)
    onehot = slot == win[:, None]
    sample = jnp.sum(jnp.where(onehot, idxs, 0), axis=1)
    out_ref[...] = sample[:, None]


def _run_finish(cv, ci, nc):
    return pl.pallas_call(
        _finish_kernel,
        in_specs=[
            pl.BlockSpec((ROWS, CAP), lambda: (0, 0)),
            pl.BlockSpec((ROWS, CAP), lambda: (0, 0)),
            pl.BlockSpec((ROWS, 16), lambda: (0, 0)),
        ],
        out_specs=pl.BlockSpec((ROWS, 1), lambda: (0, 0)),
        out_shape=jax.ShapeDtypeStruct((ROWS, 1), jnp.int32),
    )(cv, ci, nc)


def kernel(logits):
    cm, t = _run_stats(logits)
    cv, ci, nc = _run_candidates(logits, cm, t)
    return _run_finish(cv, ci, nc).reshape(ROWS)


# speedup vs baseline: 38.7689x; 3.2645x over previous
import functools

import jax
import jax.numpy as jnp
from jax import lax
from jax.experimental import pallas as pl
from jax.experimental.pallas import tpu as pltpu
from jax.experimental.pallas import tpu_sc as plsc

ROWS = 128
COLS = 100000
TOPK = 50
CHUNK = 128
NCHUNK = COLS // CHUNK
TAIL = COLS - NCHUNK * CHUNK
TAIL_START = NCHUNK * CHUNK
CMP = 784
CAPC = 64
CAP = 128

K1_BLOCK_ROWS = 8
K1_GRID = ROWS // K1_BLOCK_ROWS

NC = 2
NS = 16
NW = NC * NS
ROWS_PER_W = ROWS // NW


def _monotone_u32(x):
    b = lax.bitcast_convert_type(x, jnp.uint32)
    neg = (b >> jnp.uint32(31)) == jnp.uint32(1)
    return jnp.where(neg, ~b, b | jnp.uint32(0x80000000))


def _u32_to_f32(u):
    was_pos = (u >> jnp.uint32(31)) == jnp.uint32(1)
    b = jnp.where(was_pos, u & jnp.uint32(0x7FFFFFFF), ~u)
    return lax.bitcast_convert_type(b, jnp.float32)


def _kth_largest_key(ukey, k, batch):
    w = ukey.shape[1]
    rep = w // 128
    ones = jnp.ones((w, 128), jnp.float32)
    kf = jnp.float32(k)
    t = jnp.zeros((batch, 128), jnp.uint32)
    for bit in range(31, -1, -1):
        cand = t | jnp.uint32(1 << bit)
        cand_w = cand if rep == 1 else jnp.concatenate([cand] * rep, axis=1)
        cmpf = jnp.where(ukey >= cand_w, jnp.float32(1.0), jnp.float32(0.0))
        cnt = jax.lax.dot(cmpf, ones)
        t = jnp.where(cnt >= kf, cand, t)
    return t


def _threefry_gumbel_bits(flat_idx_u32):
    k1 = jnp.uint32(0)
    k2 = jnp.uint32(42)
    ks0, ks1 = k1, k2
    ks2 = k1 ^ k2 ^ jnp.uint32(0x1BD11BDA)

    rot1 = (13, 15, 26, 6)
    rot2 = (17, 29, 16, 24)

    def rounds(x0, x1, rots):
        for r in rots:
            x0 = x0 + x1
            x1 = (x1 << jnp.uint32(r)) | (x1 >> jnp.uint32(32 - r))
            x1 = x0 ^ x1
        return x0, x1

    x0 = jnp.zeros_like(flat_idx_u32) + ks0
    x1 = flat_idx_u32 + ks1
    x0, x1 = rounds(x0, x1, rot1)
    x0, x1 = x0 + ks1, x1 + ks2 + jnp.uint32(1)
    x0, x1 = rounds(x0, x1, rot2)
    x0, x1 = x0 + ks2, x1 + ks0 + jnp.uint32(2)
    x0, x1 = rounds(x0, x1, rot1)
    x0, x1 = x0 + ks0, x1 + ks1 + jnp.uint32(3)
    x0, x1 = rounds(x0, x1, rot2)
    x0, x1 = x0 + ks1, x1 + ks2 + jnp.uint32(4)
    x0, x1 = rounds(x0, x1, rot1)
    x0, x1 = x0 + ks2, x1 + ks0 + jnp.uint32(5)
    return x0 ^ x1


def _gumbel_from_bits(bits):
    fb = (bits >> jnp.uint32(9)) | jnp.uint32(0x3F800000)
    f = lax.bitcast_convert_type(fb, jnp.float32) - jnp.float32(1.0)
    tiny = jnp.float32(1.1754943508222875e-38)
    u = f * (jnp.float32(1.0) - tiny) + tiny
    u = jnp.maximum(tiny, u)
    return -jnp.log(-jnp.log(u))


def _stats_kernel(x_ref, cm_ref, t_ref):
    x = x_ref[...]
    xa = x[:, : NCHUNK * CHUNK].reshape(K1_BLOCK_ROWS, NCHUNK, CHUNK)
    cm = jnp.max(xa, axis=2)
    ukey = _monotone_u32(cm)
    zpad = jnp.zeros((K1_BLOCK_ROWS, 896 - NCHUNK), jnp.uint32)
    t = _kth_largest_key(
        jnp.concatenate([ukey, zpad], axis=1), TOPK, K1_BLOCK_ROWS)
    pad = jnp.full((K1_BLOCK_ROWS, CMP - NCHUNK), -jnp.inf, jnp.float32)
    cm_ref[...] = jnp.concatenate([cm, pad], axis=1)
    t_ref[...] = _u32_to_f32(t[:, :16])


def _run_stats(logits):
    return pl.pallas_call(
        _stats_kernel,
        grid=(K1_GRID,),
        in_specs=[pl.BlockSpec((K1_BLOCK_ROWS, COLS), lambda i: (i, 0))],
        out_specs=[
            pl.BlockSpec((K1_BLOCK_ROWS, CMP), lambda i: (i, 0)),
            pl.BlockSpec((K1_BLOCK_ROWS, 16), lambda i: (i, 0)),
        ],
        out_shape=[
            jax.ShapeDtypeStruct((ROWS, CMP), jnp.float32),
            jax.ShapeDtypeStruct((ROWS, 16), jnp.float32),
        ],
    )(logits)


def _candidates_kernel(x_hbm, cm_hbm, t_hbm, cv_hbm, ci_hbm, nc_hbm,
                       cm_v, t_v, qc_v, chunks_v, cval_v, cidx_v, tail_v,
                       nc_v, sem):
    wid = lax.axis_index("s") * NC + lax.axis_index("c")
    lanes = lax.iota(jnp.int32, 16)

    for rr in range(ROWS_PER_W):
        r = wid * ROWS_PER_W + rr

        pltpu.sync_copy(cm_hbm.at[r], cm_v)
        pltpu.sync_copy(t_hbm.at[r], t_v)
        tvec = t_v[...]

        for g in range(CAPC // 16):
            qc_v[pl.ds(g * 16, 16)] = jnp.zeros((16,), jnp.int32)
        off = jnp.int32(0)
        for g in range(CMP // 16):
            v = cm_v[pl.ds(g * 16, 16)]
            m = v >= tvec
            ids = lanes + jnp.int32(g * 16)
            w = jnp.minimum(off, CAPC)
            plsc.store_compressed(qc_v.at[pl.ds(w, 16)], ids, mask=m)
            off = off + jnp.sum(m.astype(jnp.int32))
        nq = jnp.minimum(off, CAPC)

        qgroups = [qc_v[pl.ds(g * 16, 16)] for g in range(CAPC // 16)]
        for j in range(CAPC):
            c = jnp.clip(qgroups[j // 16][j % 16], 0, NCHUNK - 1)
            pltpu.async_copy(
                x_hbm.at[r, pl.ds(c * CHUNK, CHUNK)], chunks_v.at[j], sem)
        pltpu.async_copy(
            x_hbm.at[r, pl.ds(TAIL_START, TAIL)], tail_v, sem)
        for j in range(CAPC):
            pltpu.make_async_copy(
                x_hbm.at[r, pl.ds(0, CHUNK)], chunks_v.at[j], sem).wait()
        pltpu.make_async_copy(
            x_hbm.at[r, pl.ds(0, TAIL)], tail_v, sem).wait()

        def chunk_body(j, noff):
            jv = jnp.full((16,), j, jnp.int32)
            cid = plsc.load_gather(qc_v, [jv])
            base = cid * jnp.int32(CHUNK)
            for s in range(CHUNK // 16):
                vals = chunks_v[j, pl.ds(s * 16, 16)]
                m = vals >= tvec
                gidx = base + jnp.int32(s * 16) + lanes
                w = jnp.minimum(noff, CAP)
                plsc.store_compressed(cval_v.at[pl.ds(w, 16)], vals, mask=m)
                plsc.store_compressed(cidx_v.at[pl.ds(w, 16)], gidx, mask=m)
                noff = noff + jnp.sum(m.astype(jnp.int32))
            return noff

        noff = lax.fori_loop(0, nq, chunk_body, jnp.int32(0))

        for s in range(TAIL // 16):
            vals = tail_v[pl.ds(s * 16, 16)]
            m = vals >= tvec
            gidx = jnp.int32(TAIL_START + s * 16) + lanes
            w = jnp.minimum(noff, CAP)
            plsc.store_compressed(cval_v.at[pl.ds(w, 16)], vals, mask=m)
            plsc.store_compressed(cidx_v.at[pl.ds(w, 16)], gidx, mask=m)
            noff = noff + jnp.sum(m.astype(jnp.int32))

        nc_v[...] = jnp.broadcast_to(jnp.minimum(noff, CAP), (16,)).astype(
            jnp.int32)
        pltpu.sync_copy(cval_v.at[pl.ds(0, CAP)], cv_hbm.at[r])
        pltpu.sync_copy(cidx_v.at[pl.ds(0, CAP)], ci_hbm.at[r])
        pltpu.sync_copy(nc_v, nc_hbm.at[r])


def _run_candidates(logits, cm, t):
    mesh = plsc.VectorSubcoreMesh(core_axis_name="c", subcore_axis_name="s")
    fn = functools.partial(
        pl.kernel,
        mesh=mesh,
        compiler_params=pltpu.CompilerParams(needs_layout_passes=False),
        out_type=[
            jax.ShapeDtypeStruct((ROWS, CAP), jnp.float32),
            jax.ShapeDtypeStruct((ROWS, CAP), jnp.int32),
            jax.ShapeDtypeStruct((ROWS, 16), jnp.int32),
        ],
        scratch_types=[
            pltpu.VMEM((CMP,), jnp.float32),
            pltpu.VMEM((16,), jnp.float32),
            pltpu.VMEM((CAPC + 16,), jnp.int32),
            pltpu.VMEM((CAPC, CHUNK), jnp.float32),
            pltpu.VMEM((CAP + 16,), jnp.float32),
            pltpu.VMEM((CAP + 16,), jnp.int32),
            pltpu.VMEM((TAIL,), jnp.float32),
            pltpu.VMEM((16,), jnp.int32),
            pltpu.SemaphoreType.DMA,
        ],
    )(_candidates_kernel)
    return fn(logits, cm, t)


def _finish_kernel(cv_ref, ci_ref, nc_ref, out_ref):
    vals = cv_ref[...]
    idxs = ci_ref[...]
    nc = nc_ref[...][:, 0:1]
    slot = lax.broadcasted_iota(jnp.int32, (ROWS, CAP), 1)
    valid = slot < nc

    ukey = jnp.where(valid, _monotone_u32(vals), jnp.uint32(0))
    kth = _u32_to_f32(_kth_largest_key(ukey, TOPK, ROWS))
    keep = valid & (vals >= kth)

    rowi = lax.broadcasted_iota(jnp.int32, (ROWS, CAP), 0)
    flat = lax.bitcast_convert_type(rowi * COLS + idxs, jnp.uint32)
    g = _gumbel_from_bits(_threefry_gumbel_bits(flat))

    score = jnp.where(keep, vals + g, jnp.float32(-jnp.inf))
    win = jnp.argmax(score, axis=1)
    onehot = slot == win[:, None]
    sample = jnp.sum(jnp.where(onehot, idxs, 0), axis=1)
    out_ref[...] = sample[:, None]


def _run_finish(cv, ci, nc):
    return pl.pallas_call(
        _finish_kernel,
        in_specs=[
            pl.BlockSpec((ROWS, CAP), lambda: (0, 0)),
            pl.BlockSpec((ROWS, CAP), lambda: (0, 0)),
            pl.BlockSpec((ROWS, 16), lambda: (0, 0)),
        ],
        out_specs=pl.BlockSpec((ROWS, 1), lambda: (0, 0)),
        out_shape=jax.ShapeDtypeStruct((ROWS, 1), jnp.int32),
    )(cv, ci, nc)


def kernel(logits):
    cm, t = _run_stats(logits)
    cv, ci, nc = _run_candidates(logits, cm, t)
    return _run_finish(cv, ci, nc).reshape(ROWS)


# speedup vs baseline: 47.6918x; 1.2302x over previous
import functools

import jax
import jax.numpy as jnp
from jax import lax
from jax.experimental import pallas as pl
from jax.experimental.pallas import tpu as pltpu
from jax.experimental.pallas import tpu_sc as plsc

ROWS = 128
COLS = 100000
TOPK = 50
CHUNK = 128
NCHUNK = COLS // CHUNK
TAIL = COLS - NCHUNK * CHUNK
TAIL_START = NCHUNK * CHUNK
CMP = 784
CAPC = 64
CAP = 128

K1_BLOCK_ROWS = 16
K1_GRID = ROWS // K1_BLOCK_ROWS

NC = 2
NS = 16
NW = NC * NS
ROWS_PER_W = ROWS // NW


def _monotone_u32(x):
    b = lax.bitcast_convert_type(x, jnp.uint32)
    neg = (b >> jnp.uint32(31)) == jnp.uint32(1)
    return jnp.where(neg, ~b, b | jnp.uint32(0x80000000))


def _u32_to_f32(u):
    was_pos = (u >> jnp.uint32(31)) == jnp.uint32(1)
    b = jnp.where(was_pos, u & jnp.uint32(0x7FFFFFFF), ~u)
    return lax.bitcast_convert_type(b, jnp.float32)


def _kth_largest_key(ukey, k, batch):
    w = ukey.shape[1]
    rep = w // 128
    ones = jnp.ones((w, 128), jnp.float32)
    kf = jnp.float32(k)
    t = jnp.zeros((batch, 128), jnp.uint32)
    for bit in range(31, -1, -1):
        cand = t | jnp.uint32(1 << bit)
        cand_w = cand if rep == 1 else jnp.concatenate([cand] * rep, axis=1)
        cmpf = jnp.where(ukey >= cand_w, jnp.float32(1.0), jnp.float32(0.0))
        cnt = jax.lax.dot(cmpf, ones)
        t = jnp.where(cnt >= kf, cand, t)
    return t


def _threefry_gumbel_bits(flat_idx_u32):
    k1 = jnp.uint32(0)
    k2 = jnp.uint32(42)
    ks0, ks1 = k1, k2
    ks2 = k1 ^ k2 ^ jnp.uint32(0x1BD11BDA)

    rot1 = (13, 15, 26, 6)
    rot2 = (17, 29, 16, 24)

    def rounds(x0, x1, rots):
        for r in rots:
            x0 = x0 + x1
            x1 = (x1 << jnp.uint32(r)) | (x1 >> jnp.uint32(32 - r))
            x1 = x0 ^ x1
        return x0, x1

    x0 = jnp.zeros_like(flat_idx_u32) + ks0
    x1 = flat_idx_u32 + ks1
    x0, x1 = rounds(x0, x1, rot1)
    x0, x1 = x0 + ks1, x1 + ks2 + jnp.uint32(1)
    x0, x1 = rounds(x0, x1, rot2)
    x0, x1 = x0 + ks2, x1 + ks0 + jnp.uint32(2)
    x0, x1 = rounds(x0, x1, rot1)
    x0, x1 = x0 + ks0, x1 + ks1 + jnp.uint32(3)
    x0, x1 = rounds(x0, x1, rot2)
    x0, x1 = x0 + ks1, x1 + ks2 + jnp.uint32(4)
    x0, x1 = rounds(x0, x1, rot1)
    x0, x1 = x0 + ks2, x1 + ks0 + jnp.uint32(5)
    return x0 ^ x1


def _gumbel_from_bits(bits):
    fb = (bits >> jnp.uint32(9)) | jnp.uint32(0x3F800000)
    f = lax.bitcast_convert_type(fb, jnp.float32) - jnp.float32(1.0)
    tiny = jnp.float32(1.1754943508222875e-38)
    u = f * (jnp.float32(1.0) - tiny) + tiny
    u = jnp.maximum(tiny, u)
    return -jnp.log(-jnp.log(u))


def _stats_kernel(x_ref, cm_ref, t_ref):
    x = x_ref[...]
    xa = x[:, : NCHUNK * CHUNK].reshape(K1_BLOCK_ROWS, NCHUNK, CHUNK)
    cm = jnp.max(xa, axis=2)
    ukey = _monotone_u32(cm)
    zpad = jnp.zeros((K1_BLOCK_ROWS, 896 - NCHUNK), jnp.uint32)
    t = _kth_largest_key(
        jnp.concatenate([ukey, zpad], axis=1), TOPK, K1_BLOCK_ROWS)
    pad = jnp.full((K1_BLOCK_ROWS, CMP - NCHUNK), -jnp.inf, jnp.float32)
    cm_ref[...] = jnp.concatenate([cm, pad], axis=1)
    t_ref[...] = _u32_to_f32(t[:, :16])


def _run_stats(logits):
    return pl.pallas_call(
        _stats_kernel,
        grid=(K1_GRID,),
        in_specs=[pl.BlockSpec((K1_BLOCK_ROWS, COLS), lambda i: (i, 0))],
        out_specs=[
            pl.BlockSpec((K1_BLOCK_ROWS, CMP), lambda i: (i, 0)),
            pl.BlockSpec((K1_BLOCK_ROWS, 16), lambda i: (i, 0)),
        ],
        out_shape=[
            jax.ShapeDtypeStruct((ROWS, CMP), jnp.float32),
            jax.ShapeDtypeStruct((ROWS, 16), jnp.float32),
        ],
    )(logits)


def _candidates_kernel(x_hbm, cm_hbm, t_hbm, cv_hbm, ci_hbm, nc_hbm,
                       cm_v, t_v, qc_v, chunks_v, cval_v, cidx_v, tail_v,
                       nc_v, sem):
    wid = lax.axis_index("s") * NC + lax.axis_index("c")
    lanes = lax.iota(jnp.int32, 16)

    for rr in range(ROWS_PER_W):
        r = wid * ROWS_PER_W + rr

        pltpu.sync_copy(cm_hbm.at[r], cm_v)
        pltpu.sync_copy(t_hbm.at[r], t_v)
        tvec = t_v[...]

        for g in range(CAPC // 16):
            qc_v[pl.ds(g * 16, 16)] = jnp.zeros((16,), jnp.int32)
        off = jnp.int32(0)
        for g in range(CMP // 16):
            v = cm_v[pl.ds(g * 16, 16)]
            m = v >= tvec
            ids = lanes + jnp.int32(g * 16)
            w = jnp.minimum(off, CAPC)
            plsc.store_compressed(qc_v.at[pl.ds(w, 16)], ids, mask=m)
            off = off + plsc.all_reduce_population_count(m)[0]
        nq = jnp.minimum(off, CAPC)

        qgroups = [qc_v[pl.ds(g * 16, 16)] for g in range(CAPC // 16)]
        for j in range(CAPC):
            c = jnp.clip(qgroups[j // 16][j % 16], 0, NCHUNK - 1)
            pltpu.async_copy(
                x_hbm.at[r, pl.ds(c * CHUNK, CHUNK)], chunks_v.at[j], sem)
        pltpu.async_copy(
            x_hbm.at[r, pl.ds(TAIL_START, TAIL)], tail_v, sem)
        for j in range(CAPC):
            pltpu.make_async_copy(
                x_hbm.at[r, pl.ds(0, CHUNK)], chunks_v.at[j], sem).wait()
        pltpu.make_async_copy(
            x_hbm.at[r, pl.ds(0, TAIL)], tail_v, sem).wait()

        def chunk_body(j, noff):
            jv = jnp.full((16,), j, jnp.int32)
            cid = plsc.load_gather(qc_v, [jv])
            base = cid * jnp.int32(CHUNK)
            for s in range(CHUNK // 16):
                vals = chunks_v[j, pl.ds(s * 16, 16)]
                m = vals >= tvec
                gidx = base + jnp.int32(s * 16) + lanes
                w = jnp.minimum(noff, CAP)
                plsc.store_compressed(cval_v.at[pl.ds(w, 16)], vals, mask=m)
                plsc.store_compressed(cidx_v.at[pl.ds(w, 16)], gidx, mask=m)
                noff = noff + plsc.all_reduce_population_count(m)[0]
            return noff

        noff = lax.fori_loop(0, nq, chunk_body, jnp.int32(0))

        for s in range(TAIL // 16):
            vals = tail_v[pl.ds(s * 16, 16)]
            m = vals >= tvec
            gidx = jnp.int32(TAIL_START + s * 16) + lanes
            w = jnp.minimum(noff, CAP)
            plsc.store_compressed(cval_v.at[pl.ds(w, 16)], vals, mask=m)
            plsc.store_compressed(cidx_v.at[pl.ds(w, 16)], gidx, mask=m)
            noff = noff + plsc.all_reduce_population_count(m)[0]

        nc_v[...] = jnp.broadcast_to(jnp.minimum(noff, CAP), (16,)).astype(
            jnp.int32)
        pltpu.sync_copy(cval_v.at[pl.ds(0, CAP)], cv_hbm.at[r])
        pltpu.sync_copy(cidx_v.at[pl.ds(0, CAP)], ci_hbm.at[r])
        pltpu.sync_copy(nc_v, nc_hbm.at[r])


def _run_candidates(logits, cm, t):
    mesh = plsc.VectorSubcoreMesh(core_axis_name="c", subcore_axis_name="s")
    fn = functools.partial(
        pl.kernel,
        mesh=mesh,
        compiler_params=pltpu.CompilerParams(needs_layout_passes=False),
        out_type=[
            jax.ShapeDtypeStruct((ROWS, CAP), jnp.float32),
            jax.ShapeDtypeStruct((ROWS, CAP), jnp.int32),
            jax.ShapeDtypeStruct((ROWS, 16), jnp.int32),
        ],
        scratch_types=[
            pltpu.VMEM((CMP,), jnp.float32),
            pltpu.VMEM((16,), jnp.float32),
            pltpu.VMEM((CAPC + 16,), jnp.int32),
            pltpu.VMEM((CAPC, CHUNK), jnp.float32),
            pltpu.VMEM((CAP + 16,), jnp.float32),
            pltpu.VMEM((CAP + 16,), jnp.int32),
            pltpu.VMEM((TAIL,), jnp.float32),
            pltpu.VMEM((16,), jnp.int32),
            pltpu.SemaphoreType.DMA,
        ],
    )(_candidates_kernel)
    return fn(logits, cm, t)


def _finish_kernel(cv_ref, ci_ref, nc_ref, out_ref):
    vals = cv_ref[...]
    idxs = ci_ref[...]
    nc = nc_ref[...][:, 0:1]
    slot = lax.broadcasted_iota(jnp.int32, (ROWS, CAP), 1)
    valid = slot < nc

    ukey = jnp.where(valid, _monotone_u32(vals), jnp.uint32(0))
    kth = _u32_to_f32(_kth_largest_key(ukey, TOPK, ROWS))
    keep = valid & (vals >= kth)

    rowi = lax.broadcasted_iota(jnp.int32, (ROWS, CAP), 0)
    flat = lax.bitcast_convert_type(rowi * COLS + idxs, jnp.uint32)
    g = _gumbel_from_bits(_threefry_gumbel_bits(flat))

    score = jnp.where(keep, vals + g, jnp.float32(-jnp.inf))
    win = jnp.argmax(score, axis=1)
    onehot = slot == win[:, None]
    sample = jnp.sum(jnp.where(onehot, idxs, 0), axis=1)
    out_ref[...] = sample[:, None]


def _run_finish(cv, ci, nc):
    return pl.pallas_call(
        _finish_kernel,
        in_specs=[
            pl.BlockSpec((ROWS, CAP), lambda: (0, 0)),
            pl.BlockSpec((ROWS, CAP), lambda: (0, 0)),
            pl.BlockSpec((ROWS, 16), lambda: (0, 0)),
        ],
        out_specs=pl.BlockSpec((ROWS, 1), lambda: (0, 0)),
        out_shape=jax.ShapeDtypeStruct((ROWS, 1), jnp.int32),
    )(cv, ci, nc)


def kernel(logits):
    cm, t = _run_stats(logits)
    cv, ci, nc = _run_candidates(logits, cm, t)
    return _run_finish(cv, ci, nc).reshape(ROWS)


# speedup vs baseline: 52.0267x; 1.0909x over previous
import functools

import jax
import jax.numpy as jnp
from jax import lax
from jax.experimental import pallas as pl
from jax.experimental.pallas import tpu as pltpu
from jax.experimental.pallas import tpu_sc as plsc

ROWS = 128
COLS = 100000
TOPK = 50
CHUNK = 128
NCHUNK = COLS // CHUNK
TAIL = COLS - NCHUNK * CHUNK
TAIL_START = NCHUNK * CHUNK
CMP = 784
CAPC = 64
CAP = 128

K1_BLOCK_ROWS = 32
K1_GRID = ROWS // K1_BLOCK_ROWS

NC = 2
NS = 16
NW = NC * NS
ROWS_PER_W = ROWS // NW


def _monotone_u32(x):
    b = lax.bitcast_convert_type(x, jnp.uint32)
    neg = (b >> jnp.uint32(31)) == jnp.uint32(1)
    return jnp.where(neg, ~b, b | jnp.uint32(0x80000000))


def _u32_to_f32(u):
    was_pos = (u >> jnp.uint32(31)) == jnp.uint32(1)
    b = jnp.where(was_pos, u & jnp.uint32(0x7FFFFFFF), ~u)
    return lax.bitcast_convert_type(b, jnp.float32)


def _kth_largest_key(ukey, k, batch):
    w = ukey.shape[1]
    rep = w // 128
    ones = jnp.ones((w, 128), jnp.float32)
    kf = jnp.float32(k)
    t = jnp.zeros((batch, 128), jnp.uint32)
    for bit in range(31, -1, -1):
        cand = t | jnp.uint32(1 << bit)
        cand_w = cand if rep == 1 else jnp.concatenate([cand] * rep, axis=1)
        cmpf = jnp.where(ukey >= cand_w, jnp.float32(1.0), jnp.float32(0.0))
        cnt = jax.lax.dot(cmpf, ones)
        t = jnp.where(cnt >= kf, cand, t)
    return t


def _threefry_gumbel_bits(flat_idx_u32):
    k1 = jnp.uint32(0)
    k2 = jnp.uint32(42)
    ks0, ks1 = k1, k2
    ks2 = k1 ^ k2 ^ jnp.uint32(0x1BD11BDA)

    rot1 = (13, 15, 26, 6)
    rot2 = (17, 29, 16, 24)

    def rounds(x0, x1, rots):
        for r in rots:
            x0 = x0 + x1
            x1 = (x1 << jnp.uint32(r)) | (x1 >> jnp.uint32(32 - r))
            x1 = x0 ^ x1
        return x0, x1

    x0 = jnp.zeros_like(flat_idx_u32) + ks0
    x1 = flat_idx_u32 + ks1
    x0, x1 = rounds(x0, x1, rot1)
    x0, x1 = x0 + ks1, x1 + ks2 + jnp.uint32(1)
    x0, x1 = rounds(x0, x1, rot2)
    x0, x1 = x0 + ks2, x1 + ks0 + jnp.uint32(2)
    x0, x1 = rounds(x0, x1, rot1)
    x0, x1 = x0 + ks0, x1 + ks1 + jnp.uint32(3)
    x0, x1 = rounds(x0, x1, rot2)
    x0, x1 = x0 + ks1, x1 + ks2 + jnp.uint32(4)
    x0, x1 = rounds(x0, x1, rot1)
    x0, x1 = x0 + ks2, x1 + ks0 + jnp.uint32(5)
    return x0 ^ x1


def _gumbel_from_bits(bits):
    fb = (bits >> jnp.uint32(9)) | jnp.uint32(0x3F800000)
    f = lax.bitcast_convert_type(fb, jnp.float32) - jnp.float32(1.0)
    tiny = jnp.float32(1.1754943508222875e-38)
    u = f * (jnp.float32(1.0) - tiny) + tiny
    u = jnp.maximum(tiny, u)
    return -jnp.log(-jnp.log(u))


def _stats_kernel(x_ref, cm_ref, t_ref):
    x = x_ref[...]
    xa = x[:, : NCHUNK * CHUNK].reshape(K1_BLOCK_ROWS, NCHUNK, CHUNK)
    cm = jnp.max(xa, axis=2)
    ukey = _monotone_u32(cm)
    zpad = jnp.zeros((K1_BLOCK_ROWS, 896 - NCHUNK), jnp.uint32)
    t = _kth_largest_key(
        jnp.concatenate([ukey, zpad], axis=1), TOPK, K1_BLOCK_ROWS)
    pad = jnp.full((K1_BLOCK_ROWS, CMP - NCHUNK), -jnp.inf, jnp.float32)
    cm_ref[...] = jnp.concatenate([cm, pad], axis=1)
    t_ref[...] = _u32_to_f32(t[:, :16])


def _run_stats(logits):
    return pl.pallas_call(
        _stats_kernel,
        grid=(K1_GRID,),
        in_specs=[pl.BlockSpec((K1_BLOCK_ROWS, COLS), lambda i: (i, 0))],
        out_specs=[
            pl.BlockSpec((K1_BLOCK_ROWS, CMP), lambda i: (i, 0)),
            pl.BlockSpec((K1_BLOCK_ROWS, 16), lambda i: (i, 0)),
        ],
        out_shape=[
            jax.ShapeDtypeStruct((ROWS, CMP), jnp.float32),
            jax.ShapeDtypeStruct((ROWS, 16), jnp.float32),
        ],
    )(logits)


def _candidates_kernel(x_hbm, cm_hbm, t_hbm, cv_hbm, ci_hbm, nc_hbm,
                       cm_v, t_v, qc_v, chunks_v, cval_v, cidx_v, tail_v,
                       nc_v, sem):
    wid = lax.axis_index("s") * NC + lax.axis_index("c")
    lanes = lax.iota(jnp.int32, 16)

    for rr in range(ROWS_PER_W):
        r = wid * ROWS_PER_W + rr

        pltpu.sync_copy(cm_hbm.at[r], cm_v)
        pltpu.sync_copy(t_hbm.at[r], t_v)
        tvec = t_v[...]

        for g in range(CAPC // 16):
            qc_v[pl.ds(g * 16, 16)] = jnp.zeros((16,), jnp.int32)
        off = jnp.int32(0)
        for g in range(CMP // 16):
            v = cm_v[pl.ds(g * 16, 16)]
            m = v >= tvec
            ids = lanes + jnp.int32(g * 16)
            w = jnp.minimum(off, CAPC)
            plsc.store_compressed(qc_v.at[pl.ds(w, 16)], ids, mask=m)
            off = off + plsc.all_reduce_population_count(m)[0]
        nq = jnp.minimum(off, CAPC)

        qgroups = [qc_v[pl.ds(g * 16, 16)] for g in range(CAPC // 16)]
        for j in range(CAPC):
            c = jnp.clip(qgroups[j // 16][j % 16], 0, NCHUNK - 1)
            pltpu.async_copy(
                x_hbm.at[r, pl.ds(c * CHUNK, CHUNK)], chunks_v.at[j], sem)
        pltpu.async_copy(
            x_hbm.at[r, pl.ds(TAIL_START, TAIL)], tail_v, sem)
        for j in range(CAPC):
            pltpu.make_async_copy(
                x_hbm.at[r, pl.ds(0, CHUNK)], chunks_v.at[j], sem).wait()
        pltpu.make_async_copy(
            x_hbm.at[r, pl.ds(0, TAIL)], tail_v, sem).wait()

        def chunk_body(j, noff):
            jv = jnp.full((16,), j, jnp.int32)
            cid = plsc.load_gather(qc_v, [jv])
            base = cid * jnp.int32(CHUNK)
            for s in range(CHUNK // 16):
                vals = chunks_v[j, pl.ds(s * 16, 16)]
                m = vals >= tvec
                gidx = base + jnp.int32(s * 16) + lanes
                w = jnp.minimum(noff, CAP)
                plsc.store_compressed(cval_v.at[pl.ds(w, 16)], vals, mask=m)
                plsc.store_compressed(cidx_v.at[pl.ds(w, 16)], gidx, mask=m)
                noff = noff + plsc.all_reduce_population_count(m)[0]
            return noff

        noff = lax.fori_loop(0, nq, chunk_body, jnp.int32(0))

        for s in range(TAIL // 16):
            vals = tail_v[pl.ds(s * 16, 16)]
            m = vals >= tvec
            gidx = jnp.int32(TAIL_START + s * 16) + lanes
            w = jnp.minimum(noff, CAP)
            plsc.store_compressed(cval_v.at[pl.ds(w, 16)], vals, mask=m)
            plsc.store_compressed(cidx_v.at[pl.ds(w, 16)], gidx, mask=m)
            noff = noff + plsc.all_reduce_population_count(m)[0]

        nc_v[...] = jnp.broadcast_to(jnp.minimum(noff, CAP), (16,)).astype(
            jnp.int32)
        pltpu.sync_copy(cval_v.at[pl.ds(0, CAP)], cv_hbm.at[r])
        pltpu.sync_copy(cidx_v.at[pl.ds(0, CAP)], ci_hbm.at[r])
        pltpu.sync_copy(nc_v, nc_hbm.at[r])


def _run_candidates(logits, cm, t):
    mesh = plsc.VectorSubcoreMesh(core_axis_name="c", subcore_axis_name="s")
    fn = functools.partial(
        pl.kernel,
        mesh=mesh,
        compiler_params=pltpu.CompilerParams(needs_layout_passes=False),
        out_type=[
            jax.ShapeDtypeStruct((ROWS, CAP), jnp.float32),
            jax.ShapeDtypeStruct((ROWS, CAP), jnp.int32),
            jax.ShapeDtypeStruct((ROWS, 16), jnp.int32),
        ],
        scratch_types=[
            pltpu.VMEM((CMP,), jnp.float32),
            pltpu.VMEM((16,), jnp.float32),
            pltpu.VMEM((CAPC + 16,), jnp.int32),
            pltpu.VMEM((CAPC, CHUNK), jnp.float32),
            pltpu.VMEM((CAP + 16,), jnp.float32),
            pltpu.VMEM((CAP + 16,), jnp.int32),
            pltpu.VMEM((TAIL,), jnp.float32),
            pltpu.VMEM((16,), jnp.int32),
            pltpu.SemaphoreType.DMA,
        ],
    )(_candidates_kernel)
    return fn(logits, cm, t)


def _finish_kernel(cv_ref, ci_ref, nc_ref, out_ref):
    vals = cv_ref[...]
    idxs = ci_ref[...]
    nc = nc_ref[...][:, 0:1]
    slot = lax.broadcasted_iota(jnp.int32, (ROWS, CAP), 1)
    valid = slot < nc

    ukey = jnp.where(valid, _monotone_u32(vals), jnp.uint32(0))
    kth = _u32_to_f32(_kth_largest_key(ukey, TOPK, ROWS))
    keep = valid & (vals >= kth)

    rowi = lax.broadcasted_iota(jnp.int32, (ROWS, CAP), 0)
    flat = lax.bitcast_convert_type(rowi * COLS + idxs, jnp.uint32)
    g = _gumbel_from_bits(_threefry_gumbel_bits(flat))

    score = jnp.where(keep, vals + g, jnp.float32(-jnp.inf))
    win = jnp.argmax(score, axis=1)
    onehot = slot == win[:, None]
    sample = jnp.sum(jnp.where(onehot, idxs, 0), axis=1)
    out_ref[...] = sample[:, None]


def _run_finish(cv, ci, nc):
    return pl.pallas_call(
        _finish_kernel,
        in_specs=[
            pl.BlockSpec((ROWS, CAP), lambda: (0, 0)),
            pl.BlockSpec((ROWS, CAP), lambda: (0, 0)),
            pl.BlockSpec((ROWS, 16), lambda: (0, 0)),
        ],
        out_specs=pl.BlockSpec((ROWS, 1), lambda: (0, 0)),
        out_shape=jax.ShapeDtypeStruct((ROWS, 1), jnp.int32),
    )(cv, ci, nc)


def kernel(logits):
    cm, t = _run_stats(logits)
    cv, ci, nc = _run_candidates(logits, cm, t)
    return _run_finish(cv, ci, nc).reshape(ROWS)


# speedup vs baseline: 53.4209x; 1.0268x over previous
import functools

import jax
import jax.numpy as jnp
from jax import lax
from jax.experimental import pallas as pl
from jax.experimental.pallas import tpu as pltpu
from jax.experimental.pallas import tpu_sc as plsc

ROWS = 128
COLS = 100000
TOPK = 50
CHUNK = 128
NCHUNK = COLS // CHUNK
TAIL = COLS - NCHUNK * CHUNK
TAIL_START = NCHUNK * CHUNK
CMP = 784
CAPC = 64
CAP = 128

K1_BLOCK_ROWS = 32
K1_GRID = ROWS // K1_BLOCK_ROWS

NC = 2
NS = 16
NW = NC * NS
ROWS_PER_W = ROWS // NW


def _monotone_u32(x):
    b = lax.bitcast_convert_type(x, jnp.uint32)
    neg = (b >> jnp.uint32(31)) == jnp.uint32(1)
    return jnp.where(neg, ~b, b | jnp.uint32(0x80000000))


def _u32_to_f32(u):
    was_pos = (u >> jnp.uint32(31)) == jnp.uint32(1)
    b = jnp.where(was_pos, u & jnp.uint32(0x7FFFFFFF), ~u)
    return lax.bitcast_convert_type(b, jnp.float32)


def _kth_largest_key(ukey, k, batch):
    w = ukey.shape[1]
    rep = w // 128
    ones = jnp.ones((w, 128), jnp.float32)
    kf = jnp.float32(k)
    t = jnp.zeros((batch, 128), jnp.uint32)
    for bit in range(31, -1, -1):
        cand = t | jnp.uint32(1 << bit)
        cand_w = cand if rep == 1 else jnp.concatenate([cand] * rep, axis=1)
        cmpf = jnp.where(ukey >= cand_w, jnp.float32(1.0), jnp.float32(0.0))
        cnt = jax.lax.dot(cmpf, ones)
        t = jnp.where(cnt >= kf, cand, t)
    return t


def _threefry_gumbel_bits(flat_idx_u32):
    k1 = jnp.uint32(0)
    k2 = jnp.uint32(42)
    ks0, ks1 = k1, k2
    ks2 = k1 ^ k2 ^ jnp.uint32(0x1BD11BDA)

    rot1 = (13, 15, 26, 6)
    rot2 = (17, 29, 16, 24)

    def rounds(x0, x1, rots):
        for r in rots:
            x0 = x0 + x1
            x1 = (x1 << jnp.uint32(r)) | (x1 >> jnp.uint32(32 - r))
            x1 = x0 ^ x1
        return x0, x1

    x0 = jnp.zeros_like(flat_idx_u32) + ks0
    x1 = flat_idx_u32 + ks1
    x0, x1 = rounds(x0, x1, rot1)
    x0, x1 = x0 + ks1, x1 + ks2 + jnp.uint32(1)
    x0, x1 = rounds(x0, x1, rot2)
    x0, x1 = x0 + ks2, x1 + ks0 + jnp.uint32(2)
    x0, x1 = rounds(x0, x1, rot1)
    x0, x1 = x0 + ks0, x1 + ks1 + jnp.uint32(3)
    x0, x1 = rounds(x0, x1, rot2)
    x0, x1 = x0 + ks1, x1 + ks2 + jnp.uint32(4)
    x0, x1 = rounds(x0, x1, rot1)
    x0, x1 = x0 + ks2, x1 + ks0 + jnp.uint32(5)
    return x0 ^ x1


def _gumbel_from_bits(bits):
    fb = (bits >> jnp.uint32(9)) | jnp.uint32(0x3F800000)
    f = lax.bitcast_convert_type(fb, jnp.float32) - jnp.float32(1.0)
    tiny = jnp.float32(1.1754943508222875e-38)
    u = f * (jnp.float32(1.0) - tiny) + tiny
    u = jnp.maximum(tiny, u)
    return -jnp.log(-jnp.log(u))


def _stats_kernel(x_ref, cm_ref, t_ref):
    x = x_ref[...]
    xa = x[:, : NCHUNK * CHUNK].reshape(K1_BLOCK_ROWS, NCHUNK, CHUNK)
    cm = jnp.max(xa, axis=2)
    ukey = _monotone_u32(cm)
    zpad = jnp.zeros((K1_BLOCK_ROWS, 896 - NCHUNK), jnp.uint32)
    t = _kth_largest_key(
        jnp.concatenate([ukey, zpad], axis=1), TOPK, K1_BLOCK_ROWS)
    pad = jnp.full((K1_BLOCK_ROWS, CMP - NCHUNK), -jnp.inf, jnp.float32)
    cm_ref[...] = jnp.concatenate([cm, pad], axis=1)
    t_ref[...] = _u32_to_f32(t[:, :16])


def _run_stats(logits):
    return pl.pallas_call(
        _stats_kernel,
        grid=(K1_GRID,),
        in_specs=[pl.BlockSpec((K1_BLOCK_ROWS, COLS), lambda i: (i, 0))],
        out_specs=[
            pl.BlockSpec((K1_BLOCK_ROWS, CMP), lambda i: (i, 0)),
            pl.BlockSpec((K1_BLOCK_ROWS, 16), lambda i: (i, 0)),
        ],
        out_shape=[
            jax.ShapeDtypeStruct((ROWS, CMP), jnp.float32),
            jax.ShapeDtypeStruct((ROWS, 16), jnp.float32),
        ],
    )(logits)


def _candidates_kernel(x_hbm, cm_hbm, t_hbm, cv_hbm, ci_hbm, nc_hbm,
                       cm_v, t_v, qc_v, chunks_v, cval_v, cidx_v, tail_v,
                       nc_v, sem):
    wid = lax.axis_index("s") * NC + lax.axis_index("c")
    lanes = lax.iota(jnp.int32, 16)

    nqs = []
    tvecs = []
    for rr in range(ROWS_PER_W):
        r = wid * ROWS_PER_W + rr
        pltpu.sync_copy(cm_hbm.at[r], cm_v)
        pltpu.sync_copy(t_hbm.at[r], t_v)
        tvec = t_v[...]
        tvecs.append(tvec)

        for g in range(CAPC // 16):
            qc_v[pl.ds(rr * (CAPC + 16) + g * 16, 16)] = jnp.zeros(
                (16,), jnp.int32)
        off = jnp.int32(0)
        for g in range(CMP // 16):
            v = cm_v[pl.ds(g * 16, 16)]
            m = v >= tvec
            ids = lanes + jnp.int32(g * 16)
            w = rr * (CAPC + 16) + jnp.minimum(off, CAPC)
            plsc.store_compressed(qc_v.at[pl.ds(w, 16)], ids, mask=m)
            off = off + plsc.all_reduce_population_count(m)[0]
        nq = jnp.minimum(off, CAPC)
        nqs.append(nq)

        def fire_body(j, acc, rr=rr, r=r):
            jv = jnp.full((16,), rr * (CAPC + 16), jnp.int32) + j
            c = jnp.clip(plsc.load_gather(qc_v, [jv])[0], 0, NCHUNK - 1)
            pltpu.async_copy(
                x_hbm.at[r, pl.ds(c * CHUNK, CHUNK)],
                chunks_v.at[rr * CAPC + j], sem)
            return acc

        lax.fori_loop(0, nq, fire_body, jnp.int32(0))
        pltpu.async_copy(
            x_hbm.at[r, pl.ds(TAIL_START, TAIL)],
            tail_v.at[rr], sem)

    for rr in range(ROWS_PER_W):
        r = wid * ROWS_PER_W + rr
        tvec = tvecs[rr]
        nq = nqs[rr]
        def drain_body(j, acc, rr=rr, r=r):
            pltpu.make_async_copy(
                x_hbm.at[r, pl.ds(0, CHUNK)],
                chunks_v.at[rr * CAPC + j], sem).wait()
            return acc

        lax.fori_loop(0, nq, drain_body, jnp.int32(0))
        pltpu.make_async_copy(
            x_hbm.at[r, pl.ds(0, TAIL)],
            tail_v.at[rr], sem).wait()

        def chunk_body(j, noff, rr=rr, tvec=tvec):
            jv = jnp.full((16,), rr * (CAPC + 16), jnp.int32) + j
            cid = plsc.load_gather(qc_v, [jv])
            base = cid * jnp.int32(CHUNK)
            for s in range(CHUNK // 16):
                vals = chunks_v[rr * CAPC + j, pl.ds(s * 16, 16)]
                m = vals >= tvec
                gidx = base + jnp.int32(s * 16) + lanes
                w = jnp.minimum(noff, CAP)
                plsc.store_compressed(cval_v.at[pl.ds(w, 16)], vals, mask=m)
                plsc.store_compressed(cidx_v.at[pl.ds(w, 16)], gidx, mask=m)
                noff = noff + plsc.all_reduce_population_count(m)[0]
            return noff

        noff = lax.fori_loop(0, nq, chunk_body, jnp.int32(0))

        for s in range(TAIL // 16):
            vals = tail_v[rr, pl.ds(s * 16, 16)]
            m = vals >= tvec
            gidx = jnp.int32(TAIL_START + s * 16) + lanes
            w = jnp.minimum(noff, CAP)
            plsc.store_compressed(cval_v.at[pl.ds(w, 16)], vals, mask=m)
            plsc.store_compressed(cidx_v.at[pl.ds(w, 16)], gidx, mask=m)
            noff = noff + plsc.all_reduce_population_count(m)[0]

        nc_v[...] = jnp.broadcast_to(jnp.minimum(noff, CAP), (16,)).astype(
            jnp.int32)
        pltpu.sync_copy(cval_v.at[pl.ds(0, CAP)], cv_hbm.at[r])
        pltpu.sync_copy(cidx_v.at[pl.ds(0, CAP)], ci_hbm.at[r])
        pltpu.sync_copy(nc_v, nc_hbm.at[r])


def _run_candidates(logits, cm, t):
    mesh = plsc.VectorSubcoreMesh(core_axis_name="c", subcore_axis_name="s")
    fn = functools.partial(
        pl.kernel,
        mesh=mesh,
        compiler_params=pltpu.CompilerParams(needs_layout_passes=False),
        out_type=[
            jax.ShapeDtypeStruct((ROWS, CAP), jnp.float32),
            jax.ShapeDtypeStruct((ROWS, CAP), jnp.int32),
            jax.ShapeDtypeStruct((ROWS, 16), jnp.int32),
        ],
        scratch_types=[
            pltpu.VMEM((CMP,), jnp.float32),
            pltpu.VMEM((16,), jnp.float32),
            pltpu.VMEM((ROWS_PER_W * (CAPC + 16),), jnp.int32),
            pltpu.VMEM((ROWS_PER_W * CAPC, CHUNK), jnp.float32),
            pltpu.VMEM((CAP + 16,), jnp.float32),
            pltpu.VMEM((CAP + 16,), jnp.int32),
            pltpu.VMEM((ROWS_PER_W, TAIL), jnp.float32),
            pltpu.VMEM((16,), jnp.int32),
            pltpu.SemaphoreType.DMA,
        ],
    )(_candidates_kernel)
    return fn(logits, cm, t)


def _finish_kernel(cv_ref, ci_ref, nc_ref, out_ref):
    vals = cv_ref[...]
    idxs = ci_ref[...]
    nc = nc_ref[...][:, 0:1]
    slot = lax.broadcasted_iota(jnp.int32, (ROWS, CAP), 1)
    valid = slot < nc

    ukey = jnp.where(valid, _monotone_u32(vals), jnp.uint32(0))
    kth = _u32_to_f32(_kth_largest_key(ukey, TOPK, ROWS))
    keep = valid & (vals >= kth)

    rowi = lax.broadcasted_iota(jnp.int32, (ROWS, CAP), 0)
    flat = lax.bitcast_convert_type(rowi * COLS + idxs, jnp.uint32)
    g = _gumbel_from_bits(_threefry_gumbel_bits(flat))

    score = jnp.where(keep, vals + g, jnp.float32(-jnp.inf))
    win = jnp.argmax(score, axis=1)
    onehot = slot == win[:, None]
    sample = jnp.sum(jnp.where(onehot, idxs, 0), axis=1)
    out_ref[...] = sample[:, None]


def _run_finish(cv, ci, nc):
    return pl.pallas_call(
        _finish_kernel,
        in_specs=[
            pl.BlockSpec((ROWS, CAP), lambda: (0, 0)),
            pl.BlockSpec((ROWS, CAP), lambda: (0, 0)),
            pl.BlockSpec((ROWS, 16), lambda: (0, 0)),
        ],
        out_specs=pl.BlockSpec((ROWS, 1), lambda: (0, 0)),
        out_shape=jax.ShapeDtypeStruct((ROWS, 1), jnp.int32),
    )(cv, ci, nc)


def kernel(logits):
    cm, t = _run_stats(logits)
    cv, ci, nc = _run_candidates(logits, cm, t)
    return _run_finish(cv, ci, nc).reshape(ROWS)
